# baseline (speedup 1.0000x reference)
import functools

import jax
import jax.numpy as jnp
from jax.experimental import pallas as pl
from jax.experimental.pallas import tpu as pltpu

BINS_ = 10
NST = 16
BC = 256


def _part(x, labels):
    s = jnp.sum(jnp.exp(x), axis=0, keepdims=True)
    rows = jax.lax.broadcasted_iota(jnp.int32, x.shape, 0)
    xt = jnp.sum(jnp.where(rows == labels, x, 0.0), axis=0, keepdims=True)
    p = jnp.exp(xt) / s
    bin_raw = jnp.floor((1.0 - p) * BINS_).astype(jnp.int32)
    sel = (bin_raw >= 0) & (bin_raw < BINS_)
    logp = xt - jnp.log(s)
    return bin_raw, sel, logp


def _t_kernel(*args, nsteps):
    i = pl.program_id(0)
    xs = args[:NST]
    ts = args[NST:2 * NST]
    out_ref = args[2 * NST]
    acc_ref = args[2 * NST + 1]

    @pl.when(i == 0)
    def _init():
        acc_ref[...] = jnp.zeros_like(acc_ref)

    parts = [_part(x[...], t[...]) for x, t in zip(xs, ts)]
    cnts = []
    slogs = []
    for b in range(BINS_):
        ms = [(bb == b) & ss for bb, ss, _ in parts]
        c = sum(jnp.sum(m.astype(jnp.float32), keepdims=True) for m in ms)
        sl = sum(jnp.sum(jnp.where(m, ll, 0.0), keepdims=True)
                 for m, (_, _, ll) in zip(ms, parts))
        cnts.append(c.reshape(1, 1))
        slogs.append(sl.reshape(1, 1))
    acc_ref[0:1, :] += jnp.concatenate(cnts, axis=1)
    acc_ref[1:2, :] += jnp.concatenate(slogs, axis=1)

    @pl.when(i == nsteps - 1)
    def _fin():
        counts = acc_ref[0:1, :]
        slog = acc_ref[1:2, :]
        nonempty = counts > 0
        n = jnp.sum(nonempty.astype(jnp.float32), keepdims=True)
        per_bin = jnp.where(nonempty, slog / jnp.maximum(counts, 1.0), 0.0)
        out_ref[...] = (-jnp.sum(per_bin, keepdims=True)
                        / jnp.maximum(n, 1.0))


def kernel(y_pred, y_true):
    n, c = y_pred.shape
    xT = y_pred.T
    tl = y_true.reshape(1, n)
    nsteps = n // (BC * NST)

    def xmap(k):
        return lambda i: (0, i + k * nsteps)

    out = pl.pallas_call(
        functools.partial(_t_kernel, nsteps=nsteps),
        grid=(nsteps,),
        in_specs=([pl.BlockSpec((c, BC), xmap(k)) for k in range(NST)]
                  + [pl.BlockSpec((1, BC), xmap(k)) for k in range(NST)]),
        out_specs=pl.BlockSpec((1, 1), lambda i: (0, 0)),
        out_shape=jax.ShapeDtypeStruct((1, 1), jnp.float32),
        scratch_shapes=[pltpu.VMEM((2, BINS_), jnp.float32)],
    )(*([xT] * NST), *([tl] * NST))
    return out[0, 0]


# speedup vs baseline: 1.0511x; 1.0511x over previous
import functools

import jax
import jax.numpy as jnp
from jax.experimental import pallas as pl
from jax.experimental.pallas import tpu as pltpu

BINS_ = 10
BC = 512


def _part(x, labels):
    s = jnp.sum(jnp.exp(x), axis=0, keepdims=True)
    rows = jax.lax.broadcasted_iota(jnp.int32, x.shape, 0)
    xt = jnp.sum(jnp.where(rows == labels, x, 0.0), axis=0, keepdims=True)
    p = jnp.exp(xt) / s
    bin_raw = jnp.floor((1.0 - p) * BINS_).astype(jnp.int32)
    sel = (bin_raw >= 0) & (bin_raw < BINS_)
    logp = xt - jnp.log(s)
    return bin_raw, sel, logp


def _t_kernel(x0_ref, x1_ref, x2_ref, x3_ref, x4_ref, x5_ref, x6_ref, x7_ref,
              t0_ref, t1_ref, t2_ref, t3_ref, t4_ref, t5_ref, t6_ref, t7_ref,
              out_ref, acc_ref, *, nsteps):
    i = pl.program_id(0)

    @pl.when(i == 0)
    def _init():
        acc_ref[...] = jnp.zeros_like(acc_ref)

    parts = [_part(x0_ref[...], t0_ref[...]), _part(x1_ref[...], t1_ref[...]),
             _part(x2_ref[...], t2_ref[...]), _part(x3_ref[...], t3_ref[...]),
             _part(x4_ref[...], t4_ref[...]), _part(x5_ref[...], t5_ref[...]),
             _part(x6_ref[...], t6_ref[...]), _part(x7_ref[...], t7_ref[...])]
    cnts = []
    slogs = []
    for b in range(BINS_):
        ms = [(bb == b) & ss for bb, ss, _ in parts]
        c = sum(jnp.sum(m.astype(jnp.float32), keepdims=True) for m in ms)
        sl = sum(jnp.sum(jnp.where(m, ll, 0.0), keepdims=True)
                 for m, (_, _, ll) in zip(ms, parts))
        cnts.append(c.reshape(1, 1))
        slogs.append(sl.reshape(1, 1))
    acc_ref[0:1, :] += jnp.concatenate(cnts, axis=1)
    acc_ref[1:2, :] += jnp.concatenate(slogs, axis=1)

    @pl.when(i == nsteps - 1)
    def _fin():
        counts = acc_ref[0:1, :]
        slog = acc_ref[1:2, :]
        nonempty = counts > 0
        n = jnp.sum(nonempty.astype(jnp.float32), keepdims=True)
        per_bin = jnp.where(nonempty, slog / jnp.maximum(counts, 1.0), 0.0)
        out_ref[...] = (-jnp.sum(per_bin, keepdims=True)
                        / jnp.maximum(n, 1.0))


def kernel(y_pred, y_true):
    n, c = y_pred.shape
    xT = y_pred.T
    tl = y_true.reshape(1, n)
    nsteps = n // (BC * 8)
    out = pl.pallas_call(
        functools.partial(_t_kernel, nsteps=nsteps),
        grid=(nsteps,),
        in_specs=(
            [pl.BlockSpec((c, BC), functools.partial(
                lambda k, i: (0, i + k * nsteps), k)) for k in range(8)]
            + [pl.BlockSpec((1, BC), functools.partial(
                lambda k, i: (0, i + k * nsteps), k)) for k in range(8)]),
        out_specs=pl.BlockSpec((1, 1), lambda i: (0, 0)),
        out_shape=jax.ShapeDtypeStruct((1, 1), jnp.float32),
        scratch_shapes=[pltpu.VMEM((2, BINS_), jnp.float32)],
    )(*([xT] * 8), *([tl] * 8))
    return out[0, 0]
